# Initial kernel scaffold; baseline (speedup 1.0000x reference)
import jax
import jax.numpy as jnp
from jax import lax
from jax.experimental import pallas as pl
from jax.experimental.pallas import tpu as pltpu
from jax.experimental.pallas import tpu_sc as plsc

B = 4
S = 4096
NQ = 64
D = 2048
NW = 32
ROWS_PER_W = (B * S) // NW
CHUNK = 16
NCHUNKS = ROWS_PER_W // CHUNK
WPB = NW // B


def _body(ids_hbm, vis_hbm, img_hbm, table_hbm, out_hbm,
          idx_v, img_v, buf, rowbuf, sem):
    wid = lax.axis_index("s") * 2 + lax.axis_index("c")
    base = wid * ROWS_PER_W

    pltpu.sync_copy(ids_hbm.at[pl.ds(base, ROWS_PER_W)], idx_v)

    def gather_chunk(i, carry):
        row0 = base + i * CHUNK
        pltpu.async_copy(table_hbm.at[idx_v.at[pl.ds(i * CHUNK, CHUNK)]],
                         buf, sem).wait()
        pltpu.sync_copy(buf, out_hbm.at[pl.ds(row0, CHUNK)])
        return carry

    lax.fori_loop(0, NCHUNKS, gather_chunk, 0)

    b = wid // WPB
    s0 = (wid % WPB) * ROWS_PER_W
    pltpu.sync_copy(img_hbm.at[pl.ds(b * NQ, NQ)], img_v)

    def vis_one(e, carry):
        pos = img_v[e]
        hit = jnp.logical_and(pos >= s0, pos < s0 + ROWS_PER_W)

        @pl.when(hit)
        def _():
            pltpu.sync_copy(vis_hbm.at[b * NQ + e], rowbuf)
            pltpu.sync_copy(rowbuf, out_hbm.at[b * S + pos])

        return carry

    lax.fori_loop(0, NQ, vis_one, 0)


@jax.jit
def kernel(input_ids, vision_hidden_states, image_indices, embed_table):
    ids = input_ids.reshape(B * S).astype(jnp.int32)
    vis = vision_hidden_states.reshape(B * NQ, D)
    img = image_indices.reshape(B * NQ).astype(jnp.int32)

    mesh = plsc.VectorSubcoreMesh(core_axis_name="c", subcore_axis_name="s")
    out = pl.kernel(
        _body,
        out_type=jax.ShapeDtypeStruct((B * S, D), jnp.float32),
        mesh=mesh,
        scratch_types=[
            pltpu.VMEM((ROWS_PER_W,), jnp.int32),
            pltpu.VMEM((NQ,), jnp.int32),
            pltpu.VMEM((CHUNK, D), jnp.float32),
            pltpu.VMEM((D,), jnp.float32),
            pltpu.SemaphoreType.DMA,
        ],
    )(ids, vis, img, embed_table)
    return out.reshape(B, S, D)


# speedup vs baseline: 1.6876x; 1.6876x over previous
import jax
import jax.numpy as jnp
from jax import lax
from jax.experimental import pallas as pl
from jax.experimental.pallas import tpu as pltpu
from jax.experimental.pallas import tpu_sc as plsc

B = 4
S = 4096
NQ = 64
D = 2048
NW = 32
ROWS_PER_W = (B * S) // NW
CHUNK = 16
NCHUNKS = ROWS_PER_W // CHUNK
WPB = NW // B


def _body(ids_hbm, vis_hbm, img_hbm, table_hbm, out_hbm,
          idx_v, img_v, buf, rowbuf, sem):
    wid = lax.axis_index("s") * 2 + lax.axis_index("c")
    base = wid * ROWS_PER_W

    pltpu.sync_copy(ids_hbm.at[pl.ds(base, ROWS_PER_W)], idx_v)

    def gather_chunk(i, carry):
        row0 = base + i * CHUNK
        pltpu.async_copy(table_hbm.at[idx_v.at[pl.ds(i * CHUNK, CHUNK)]],
                         buf, sem).wait()
        pltpu.sync_copy(buf, out_hbm.at[pl.ds(row0, CHUNK)])
        return carry

    lax.fori_loop(0, NCHUNKS, gather_chunk, 0)

    b = wid // WPB
    s0 = (wid % WPB) * ROWS_PER_W
    pltpu.sync_copy(img_hbm.at[pl.ds(b * NQ, NQ)], img_v.at[pl.ds(0, NQ)])
    pltpu.sync_copy(img_hbm.at[pl.ds(b * NQ + NQ - 16, 16)],
                    img_v.at[pl.ds(NQ, 16)])

    def vis_one(e, carry):
        pos = img_v[pl.ds(e, 16)][0]
        hit = jnp.logical_and(pos >= s0, pos < s0 + ROWS_PER_W)

        @pl.when(hit)
        def _():
            pltpu.sync_copy(vis_hbm.at[b * NQ + e], rowbuf)
            pltpu.sync_copy(rowbuf, out_hbm.at[b * S + pos])

        return carry

    lax.fori_loop(0, NQ, vis_one, 0)


@jax.jit
def kernel(input_ids, vision_hidden_states, image_indices, embed_table):
    ids = input_ids.reshape(B * S).astype(jnp.int32)
    vis = vision_hidden_states.reshape(B * NQ, D)
    img = image_indices.reshape(B * NQ).astype(jnp.int32)

    mesh = plsc.VectorSubcoreMesh(core_axis_name="c", subcore_axis_name="s")
    out = pl.kernel(
        _body,
        out_type=jax.ShapeDtypeStruct((B * S, D), jnp.float32),
        mesh=mesh,
        scratch_types=[
            pltpu.VMEM((ROWS_PER_W,), jnp.int32),
            pltpu.VMEM((NQ + 16,), jnp.int32),
            pltpu.VMEM((CHUNK, D), jnp.float32),
            pltpu.VMEM((D,), jnp.float32),
            pltpu.SemaphoreType.DMA,
        ],
    )(ids, vis, img, embed_table)
    return out.reshape(B, S, D)


# speedup vs baseline: 2.0060x; 1.1887x over previous
import jax
import jax.numpy as jnp
from jax import lax
from jax.experimental import pallas as pl
from jax.experimental.pallas import tpu as pltpu
from jax.experimental.pallas import tpu_sc as plsc

B = 4
S = 4096
NQ = 64
D = 2048
NW = 32
ROWS_PER_W = (B * S) // NW
CHUNK = 16
NCHUNKS = ROWS_PER_W // CHUNK
WPB = NW // B


def _body(ids_hbm, vis_hbm, img_hbm, table_hbm, out_hbm,
          idx_v, img_v, buf0, buf1, rowbuf, gsem0, gsem1, wsem0, wsem1):
    wid = lax.axis_index("s") * 2 + lax.axis_index("c")
    base = wid * ROWS_PER_W

    bufs = (buf0, buf1)
    gsems = (gsem0, gsem1)
    wsems = (wsem0, wsem1)

    pltpu.sync_copy(ids_hbm.at[pl.ds(base, ROWS_PER_W)], idx_v)

    def issue_gather(i, b):
        pltpu.async_copy(table_hbm.at[idx_v.at[pl.ds(i * CHUNK, CHUNK)]],
                         bufs[b], gsems[b])

    issue_gather(0, 0)
    issue_gather(1, 1)

    @pl.loop(0, NCHUNKS, step=2)
    def _pipe(g):
        for b in range(2):
            i = g + b
            pltpu.make_async_copy(table_hbm.at[pl.ds(0, CHUNK)],
                                  bufs[b], gsems[b]).wait()
            pltpu.async_copy(bufs[b], out_hbm.at[pl.ds(base + i * CHUNK, CHUNK)],
                             wsems[b])
            pltpu.make_async_copy(bufs[b], out_hbm.at[pl.ds(base, CHUNK)],
                                  wsems[b]).wait()

            @pl.when(i + 2 < NCHUNKS)
            def _():
                issue_gather(i + 2, b)

    b = wid // WPB
    s0 = (wid % WPB) * ROWS_PER_W
    pltpu.sync_copy(img_hbm.at[pl.ds(b * NQ, NQ)], img_v.at[pl.ds(0, NQ)])
    pltpu.sync_copy(img_hbm.at[pl.ds(b * NQ + NQ - 16, 16)],
                    img_v.at[pl.ds(NQ, 16)])

    def vis_one(e, carry):
        pos = img_v[pl.ds(e, 16)][0]
        hit = jnp.logical_and(pos >= s0, pos < s0 + ROWS_PER_W)

        @pl.when(hit)
        def _():
            pltpu.sync_copy(vis_hbm.at[b * NQ + e], rowbuf)
            pltpu.sync_copy(rowbuf, out_hbm.at[b * S + pos])

        return carry

    lax.fori_loop(0, NQ, vis_one, 0)


@jax.jit
def kernel(input_ids, vision_hidden_states, image_indices, embed_table):
    ids = input_ids.reshape(B * S).astype(jnp.int32)
    vis = vision_hidden_states.reshape(B * NQ, D)
    img = image_indices.reshape(B * NQ).astype(jnp.int32)

    mesh = plsc.VectorSubcoreMesh(core_axis_name="c", subcore_axis_name="s")
    out = pl.kernel(
        _body,
        out_type=jax.ShapeDtypeStruct((B * S, D), jnp.float32),
        mesh=mesh,
        scratch_types=[
            pltpu.VMEM((ROWS_PER_W,), jnp.int32),
            pltpu.VMEM((NQ + 16,), jnp.int32),
            pltpu.VMEM((CHUNK, D), jnp.float32),
            pltpu.VMEM((CHUNK, D), jnp.float32),
            pltpu.VMEM((D,), jnp.float32),
            pltpu.SemaphoreType.DMA,
            pltpu.SemaphoreType.DMA,
            pltpu.SemaphoreType.DMA,
            pltpu.SemaphoreType.DMA,
        ],
    )(ids, vis, img, embed_table)
    return out.reshape(B, S, D)
